# baseline (speedup 1.0000x reference)
import functools
import jax
import jax.numpy as jnp
from jax import lax
from jax.experimental import pallas as pl
from jax.experimental.pallas import tpu as pltpu
from jax.experimental.pallas import tpu_sc as plsc

N_NODES = 50000
N_OUT = 5000
N_EDGES = 1600000
OUT_BASE = N_NODES - N_OUT

NC, NS = 2, 16
NW = NC * NS
EPW = N_EDGES // NW
CHUNK = 2000
NCHUNK = EPW // CHUNK
VEC = 16
NVEC = CHUNK // VEC
UNROLL = 5
ACC = 5120

_mesh = plsc.VectorSubcoreMesh(
    core_axis_name="c", subcore_axis_name="s", num_cores=NC, num_subcores=NS
)


@functools.partial(
    pl.kernel,
    out_type=jax.ShapeDtypeStruct((NW, ACC), jnp.float32),
    mesh=_mesh,
    scratch_types=[
        pltpu.VMEM((N_NODES,), jnp.float32),
        pltpu.VMEM((ACC,), jnp.float32),
        pltpu.VMEM((CHUNK,), jnp.int32),
        pltpu.VMEM((CHUNK,), jnp.int32),
        pltpu.VMEM((CHUNK,), jnp.int32),
        pltpu.VMEM((CHUNK,), jnp.int32),
        pltpu.VMEM((CHUNK,), jnp.float32),
        pltpu.VMEM((CHUNK,), jnp.float32),
        pltpu.SemaphoreType.DMA,
        pltpu.SemaphoreType.DMA,
        pltpu.SemaphoreType.DMA,
    ],
    compiler_params=pltpu.CompilerParams(needs_layout_passes=False),
)
def _sc_partial(h_hbm, src_hbm, dst_hbm, w_hbm, out_hbm,
                h_l, acc, src_b0, src_b1, dst_b0, dst_b1, w_b0, w_b1,
                sem0, sem1, sem_h):
    wid = lax.axis_index("s") * NC + lax.axis_index("c")
    base = wid * EPW
    sems = (sem0, sem1)
    src_bufs = (src_b0, src_b1)
    dst_bufs = (dst_b0, dst_b1)
    w_bufs = (w_b0, w_b1)

    def start_chunk(ci, slot):
        off = base + ci * CHUNK
        return [
            pltpu.async_copy(src_hbm.at[pl.ds(off, CHUNK)], src_bufs[slot],
                             sems[slot]),
            pltpu.async_copy(dst_hbm.at[pl.ds(off, CHUNK)], dst_bufs[slot],
                             sems[slot]),
            pltpu.async_copy(w_hbm.at[pl.ds(off, CHUNK)], w_bufs[slot],
                             sems[slot]),
        ]

    h_dma = pltpu.async_copy(h_hbm, h_l, sem_h)
    pending = start_chunk(0, 0)

    def zero_body(i, carry):
        acc[pl.ds(i * VEC, VEC)] = jnp.zeros((VEC,), jnp.float32)
        return carry

    lax.fori_loop(0, ACC // VEC, zero_body, 0)
    h_dma.wait()

    for ci in range(NCHUNK):
        slot = ci % 2
        nxt = pending
        if ci + 1 < NCHUNK:
            nxt = start_chunk(ci + 1, 1 - slot)
        for d in pending:
            d.wait()
        pending = nxt

        def vec_body(vi, c2, _slot=slot):
            for u in range(UNROLL):
                s = vi * (VEC * UNROLL) + u * VEC
                dst = dst_bufs[_slot][pl.ds(s, VEC)]
                mask = dst >= OUT_BASE
                srcv = src_bufs[_slot][pl.ds(s, VEC)]
                wv = w_bufs[_slot][pl.ds(s, VEC)]
                h = plsc.load_gather(h_l, [srcv])
                idx = jnp.where(mask, dst - OUT_BASE, 0)
                plsc.addupdate_scatter(acc, [idx], wv * h, mask=mask)
            return c2

        lax.fori_loop(0, NVEC // UNROLL, vec_body, 0)

    pltpu.sync_copy(acc, out_hbm.at[wid])


def _tc_tail_body(p_ref, b_ref, o_ref):
    o_ref[...] = jnp.tanh(
        jnp.sum(p_ref[...], axis=0, keepdims=True) + b_ref[...]
    )


@jax.jit
def kernel(obs, h_prev, edge_weight, bias, edge_src, edge_dst):
    part = _sc_partial(h_prev, edge_src, edge_dst, edge_weight)
    bias_pad = jnp.pad(bias[OUT_BASE:], (0, ACC - N_OUT)).reshape(1, ACC)
    out = pl.pallas_call(
        _tc_tail_body,
        out_shape=jax.ShapeDtypeStruct((1, ACC), jnp.float32),
    )(part, bias_pad)
    return out.reshape(ACC)[:N_OUT]


# speedup vs baseline: 1.3261x; 1.3261x over previous
import functools
import jax
import jax.numpy as jnp
from jax import lax
from jax.experimental import pallas as pl
from jax.experimental.pallas import tpu as pltpu
from jax.experimental.pallas import tpu_sc as plsc

N_NODES = 50000
N_OUT = 5000
N_EDGES = 1600000
OUT_BASE = N_NODES - N_OUT

NC, NS = 2, 16
NW = NC * NS
EPW = N_EDGES // NW
CHUNK = 2000
NCHUNK = EPW // CHUNK
VEC = 16
NVEC = CHUNK // VEC
UNROLL = 5
ACC = 5120

_mesh = plsc.VectorSubcoreMesh(
    core_axis_name="c", subcore_axis_name="s", num_cores=NC, num_subcores=NS
)


@functools.partial(
    pl.kernel,
    out_type=jax.ShapeDtypeStruct((NW, ACC), jnp.float32),
    mesh=_mesh,
    scratch_types=[
        pltpu.VMEM((N_NODES,), jnp.float32),
        pltpu.VMEM((ACC,), jnp.float32),
        pltpu.VMEM((CHUNK,), jnp.int32),
        pltpu.VMEM((CHUNK,), jnp.int32),
        pltpu.VMEM((CHUNK,), jnp.int32),
        pltpu.VMEM((CHUNK,), jnp.int32),
        pltpu.VMEM((CHUNK,), jnp.float32),
        pltpu.VMEM((CHUNK,), jnp.float32),
        pltpu.SemaphoreType.DMA,
        pltpu.SemaphoreType.DMA,
        pltpu.SemaphoreType.DMA,
    ],
    compiler_params=pltpu.CompilerParams(needs_layout_passes=False),
)
def _sc_partial(h_hbm, src_hbm, dst_hbm, w_hbm, out_hbm,
                h_l, acc, src_b0, src_b1, dst_b0, dst_b1, w_b0, w_b1,
                sem0, sem1, sem_h):
    wid = lax.axis_index("s") * NC + lax.axis_index("c")
    base = wid * EPW
    sems = (sem0, sem1)
    src_bufs = (src_b0, src_b1)
    dst_bufs = (dst_b0, dst_b1)
    w_bufs = (w_b0, w_b1)

    def start_chunk(ci, slot):
        off = base + ci * CHUNK
        return [
            pltpu.async_copy(src_hbm.at[pl.ds(off, CHUNK)], src_bufs[slot],
                             sems[slot]),
            pltpu.async_copy(dst_hbm.at[pl.ds(off, CHUNK)], dst_bufs[slot],
                             sems[slot]),
            pltpu.async_copy(w_hbm.at[pl.ds(off, CHUNK)], w_bufs[slot],
                             sems[slot]),
        ]

    h_dma = pltpu.async_copy(h_hbm, h_l, sem_h)
    pending = start_chunk(0, 0)

    def zero_body(i, carry):
        acc[pl.ds(i * VEC, VEC)] = jnp.zeros((VEC,), jnp.float32)
        return carry

    lax.fori_loop(0, ACC // VEC, zero_body, 0)
    h_dma.wait()

    for ci in range(NCHUNK):
        slot = ci % 2
        nxt = pending
        if ci + 1 < NCHUNK:
            nxt = start_chunk(ci + 1, 1 - slot)
        for d in pending:
            d.wait()
        pending = nxt

        @plsc.parallel_loop(0, NVEC, 1, unroll=UNROLL)
        def vec_body(vi, _slot=slot):
            s = vi * VEC
            dst = dst_bufs[_slot][pl.ds(s, VEC)]
            mask = dst >= OUT_BASE
            srcv = src_bufs[_slot][pl.ds(s, VEC)]
            wv = w_bufs[_slot][pl.ds(s, VEC)]
            h = plsc.load_gather(h_l, [srcv])
            idx = jnp.where(mask, dst - OUT_BASE, 0)
            plsc.addupdate_scatter(acc, [idx], wv * h, mask=mask)

    pltpu.sync_copy(acc, out_hbm.at[wid])


def _tc_tail_body(p_ref, b_ref, o_ref):
    o_ref[...] = jnp.tanh(
        jnp.sum(p_ref[...], axis=0, keepdims=True) + b_ref[...]
    )


@jax.jit
def kernel(obs, h_prev, edge_weight, bias, edge_src, edge_dst):
    part = _sc_partial(h_prev, edge_src, edge_dst, edge_weight)
    bias_pad = jnp.pad(bias[OUT_BASE:], (0, ACC - N_OUT)).reshape(1, ACC)
    out = pl.pallas_call(
        _tc_tail_body,
        out_shape=jax.ShapeDtypeStruct((1, ACC), jnp.float32),
    )(part, bias_pad)
    return out.reshape(ACC)[:N_OUT]


# speedup vs baseline: 1.6104x; 1.2144x over previous
import functools
import jax
import jax.numpy as jnp
from jax import lax
from jax.experimental import pallas as pl
from jax.experimental.pallas import tpu as pltpu
from jax.experimental.pallas import tpu_sc as plsc

N_NODES = 50000
N_OUT = 5000
N_EDGES = 1600000
OUT_BASE = N_NODES - N_OUT

NC, NS = 2, 16
NW = NC * NS
EPW = N_EDGES // NW
CHUNK = 10000
NCHUNK = EPW // CHUNK
VEC = 16
NVEC = CHUNK // VEC
UNROLL = 5
ACC = 5120

_mesh = plsc.VectorSubcoreMesh(
    core_axis_name="c", subcore_axis_name="s", num_cores=NC, num_subcores=NS
)


@functools.partial(
    pl.kernel,
    out_type=jax.ShapeDtypeStruct((NW, ACC), jnp.float32),
    mesh=_mesh,
    scratch_types=[
        pltpu.VMEM((N_NODES,), jnp.float32),
        pltpu.VMEM((ACC,), jnp.float32),
        pltpu.VMEM((CHUNK,), jnp.int32),
        pltpu.VMEM((CHUNK,), jnp.int32),
        pltpu.VMEM((CHUNK,), jnp.int32),
        pltpu.VMEM((CHUNK,), jnp.int32),
        pltpu.VMEM((CHUNK,), jnp.float32),
        pltpu.VMEM((CHUNK,), jnp.float32),
        pltpu.SemaphoreType.DMA,
        pltpu.SemaphoreType.DMA,
        pltpu.SemaphoreType.DMA,
    ],
    compiler_params=pltpu.CompilerParams(needs_layout_passes=False),
)
def _sc_partial(h_hbm, src_hbm, dst_hbm, w_hbm, out_hbm,
                h_l, acc, src_b0, src_b1, dst_b0, dst_b1, w_b0, w_b1,
                sem0, sem1, sem_h):
    wid = lax.axis_index("s") * NC + lax.axis_index("c")
    base = wid * EPW
    sems = (sem0, sem1)
    src_bufs = (src_b0, src_b1)
    dst_bufs = (dst_b0, dst_b1)
    w_bufs = (w_b0, w_b1)

    def start_chunk(ci, slot):
        off = base + ci * CHUNK
        return [
            pltpu.async_copy(src_hbm.at[pl.ds(off, CHUNK)], src_bufs[slot],
                             sems[slot]),
            pltpu.async_copy(dst_hbm.at[pl.ds(off, CHUNK)], dst_bufs[slot],
                             sems[slot]),
            pltpu.async_copy(w_hbm.at[pl.ds(off, CHUNK)], w_bufs[slot],
                             sems[slot]),
        ]

    h_dma = pltpu.async_copy(h_hbm, h_l, sem_h)
    pending = start_chunk(0, 0)

    def zero_body(i, carry):
        acc[pl.ds(i * VEC, VEC)] = jnp.zeros((VEC,), jnp.float32)
        return carry

    lax.fori_loop(0, ACC // VEC, zero_body, 0)
    h_dma.wait()

    for ci in range(NCHUNK):
        slot = ci % 2
        nxt = pending
        if ci + 1 < NCHUNK:
            nxt = start_chunk(ci + 1, 1 - slot)
        for d in pending:
            d.wait()
        pending = nxt

        @plsc.parallel_loop(0, NVEC, 1, unroll=UNROLL)
        def vec_body(vi, _slot=slot):
            s = vi * VEC
            dst = dst_bufs[_slot][pl.ds(s, VEC)]
            mask = dst >= OUT_BASE
            srcv = src_bufs[_slot][pl.ds(s, VEC)]
            wv = w_bufs[_slot][pl.ds(s, VEC)]
            h = plsc.load_gather(h_l, [srcv])
            idx = jnp.where(mask, dst - OUT_BASE, 0)
            plsc.addupdate_scatter(acc, [idx], wv * h, mask=mask)

    pltpu.sync_copy(acc, out_hbm.at[wid])


def _tc_tail_body(p_ref, b_ref, o_ref):
    pre = jnp.sum(p_ref[...], axis=0)[:N_OUT] + b_ref[pl.ds(OUT_BASE, N_OUT)]
    o_ref[...] = jnp.tanh(pre)


@jax.jit
def kernel(obs, h_prev, edge_weight, bias, edge_src, edge_dst):
    part = _sc_partial(h_prev, edge_src, edge_dst, edge_weight)
    return pl.pallas_call(
        _tc_tail_body,
        out_shape=jax.ShapeDtypeStruct((N_OUT,), jnp.float32),
    )(part, bias)
